# baseline (speedup 1.0000x reference)
import jax
import jax.numpy as jnp
from jax.experimental import pallas as pl

_TM = 512


def _router_body(x_ref, w_ref, b_ref, o_ref):
    xb = x_ref[...].astype(jnp.bfloat16)
    acc = jax.lax.dot_general(
        xb,
        w_ref[...],
        dimension_numbers=(((1,), (1,)), ((), ())),
        preferred_element_type=jnp.float32,
    )
    o_ref[...] = acc + b_ref[...]


def kernel(hidden_states, W, b):
    tokens, hidden = hidden_states.shape
    experts = W.shape[0]
    wb = W.astype(jnp.bfloat16)
    b2 = b.reshape(1, experts)
    return pl.pallas_call(
        _router_body,
        grid=(tokens // _TM,),
        in_specs=[
            pl.BlockSpec((_TM, hidden), lambda i: (i, 0)),
            pl.BlockSpec((experts, hidden), lambda i: (0, 0)),
            pl.BlockSpec((1, experts), lambda i: (0, 0)),
        ],
        out_specs=pl.BlockSpec((_TM, experts), lambda i: (i, 0)),
        out_shape=jax.ShapeDtypeStruct((tokens, experts), jnp.float32),
    )(hidden_states, wb, b2)


# speedup vs baseline: 1.1025x; 1.1025x over previous
import jax
import jax.numpy as jnp
from jax.experimental import pallas as pl

_TM = 1024


def _router_body(x_ref, w_ref, b_ref, o_ref):
    xb = x_ref[...].astype(jnp.bfloat16)
    acc = jax.lax.dot_general(
        xb,
        w_ref[...],
        dimension_numbers=(((1,), (0,)), ((), ())),
        preferred_element_type=jnp.float32,
    )
    o_ref[...] = acc + b_ref[...]


def kernel(hidden_states, W, b):
    tokens, hidden = hidden_states.shape
    experts = W.shape[0]
    wt = W.T.astype(jnp.bfloat16)
    b2 = b.reshape(1, experts)
    return pl.pallas_call(
        _router_body,
        grid=(tokens // _TM,),
        in_specs=[
            pl.BlockSpec((_TM, hidden), lambda i: (i, 0)),
            pl.BlockSpec((hidden, experts), lambda i: (0, 0)),
            pl.BlockSpec((1, experts), lambda i: (0, 0)),
        ],
        out_specs=pl.BlockSpec((_TM, experts), lambda i: (i, 0)),
        out_shape=jax.ShapeDtypeStruct((tokens, experts), jnp.float32),
    )(hidden_states, wt, b2)


# speedup vs baseline: 1.1043x; 1.0016x over previous
import jax
import jax.numpy as jnp
from jax.experimental import pallas as pl
from jax.experimental.pallas import tpu as pltpu

_TM = 1024


def _router_body(x_ref, w_ref, b_ref, o_ref):
    xb = x_ref[...].astype(jnp.bfloat16)
    acc = jax.lax.dot_general(
        xb,
        w_ref[...],
        dimension_numbers=(((1,), (1,)), ((), ())),
        preferred_element_type=jnp.float32,
    )
    o_ref[...] = acc + b_ref[...]


def kernel(hidden_states, W, b):
    tokens, hidden = hidden_states.shape
    experts = W.shape[0]
    wb = W.astype(jnp.bfloat16)
    b2 = b.reshape(1, experts)
    return pl.pallas_call(
        _router_body,
        grid=(tokens // _TM,),
        in_specs=[
            pl.BlockSpec((_TM, hidden), lambda i: (i, 0)),
            pl.BlockSpec((experts, hidden), lambda i: (0, 0)),
            pl.BlockSpec((1, experts), lambda i: (0, 0)),
        ],
        out_specs=pl.BlockSpec((_TM, experts), lambda i: (i, 0)),
        out_shape=jax.ShapeDtypeStruct((tokens, experts), jnp.float32),
        compiler_params=pltpu.CompilerParams(
            vmem_limit_bytes=100 * 1024 * 1024,
        ),
    )(hidden_states, wb, b2)


# speedup vs baseline: 1.1366x; 1.0293x over previous
import jax
import jax.numpy as jnp
from jax.experimental import pallas as pl
from jax.experimental.pallas import tpu as pltpu

_TM = 1024


def _router_body(x_ref, w_ref, b_ref, o_ref):
    xb = x_ref[...].astype(jnp.bfloat16)
    wb = w_ref[...].astype(jnp.bfloat16)
    acc = jax.lax.dot_general(
        xb,
        wb,
        dimension_numbers=(((1,), (1,)), ((), ())),
        preferred_element_type=jnp.float32,
    )
    o_ref[...] = acc + b_ref[...]


def kernel(hidden_states, W, b):
    tokens, hidden = hidden_states.shape
    experts = W.shape[0]
    b2 = b.reshape(1, experts)
    return pl.pallas_call(
        _router_body,
        grid=(tokens // _TM,),
        in_specs=[
            pl.BlockSpec((_TM, hidden), lambda i: (i, 0)),
            pl.BlockSpec((experts, hidden), lambda i: (0, 0)),
            pl.BlockSpec((1, experts), lambda i: (0, 0)),
        ],
        out_specs=pl.BlockSpec((_TM, experts), lambda i: (i, 0)),
        out_shape=jax.ShapeDtypeStruct((tokens, experts), jnp.float32),
        compiler_params=pltpu.CompilerParams(
            vmem_limit_bytes=100 * 1024 * 1024,
        ),
    )(hidden_states, W, b2)
